# baseline (speedup 1.0000x reference)
import functools

import jax
import jax.numpy as jnp
from jax.experimental import pallas as pl
from jax.experimental.pallas import tpu as pltpu
from jax.experimental.pallas import tpu_sc as plsc

N_NODES = 10000
N_EDGES = 320000
D = 100
DIM = 64

DP = 128
NODE_PAD = 10240
DUMMY = NODE_PAD - 1
N_SUBCORES = 16
CHUNK = 56
CHUNKS_PER_TILE = 360
E_PAD = N_SUBCORES * CHUNK * CHUNKS_PER_TILE
SPMEM_ROWS = 10112
SCAT_DUMMY = N_NODES
ROWS_PER_TILE = SPMEM_ROWS // N_SUBCORES
_COPY_CHUNKS = [(i * CHUNK, CHUNK) for i in range(11)] + [(616, 16)]

_f32 = jnp.float32
_bf16 = jnp.bfloat16


def _dot(a, b):
    return jax.lax.dot_general(a, b, (((1,), (0,)), ((), ())),
                               preferred_element_type=_f32)


def _proj_body(x_ref, wi_ref, wj_ref, b_ref, xi_ref, xj_ref):
    x = x_ref[...]
    xi_ref[...] = _dot(x, wi_ref[...]) + b_ref[0:1, :]
    xj_ref[...] = _dot(x, wj_ref[...])


def _proj(x_bf, wi_t, wj_t, b_pad):
    n = x_bf.shape[0]
    blk = 256
    return pl.pallas_call(
        _proj_body,
        grid=(n // blk,),
        in_specs=[
            pl.BlockSpec((blk, DP), lambda i: (i, 0)),
            pl.BlockSpec((DP, DP), lambda i: (0, 0)),
            pl.BlockSpec((DP, DP), lambda i: (0, 0)),
            pl.BlockSpec((8, DP), lambda i: (0, 0)),
        ],
        out_specs=[
            pl.BlockSpec((blk, DP), lambda i: (i, 0)),
            pl.BlockSpec((blk, DP), lambda i: (i, 0)),
        ],
        out_shape=[
            jax.ShapeDtypeStruct((n, DP), _f32),
            jax.ShapeDtypeStruct((n, DP), _f32),
        ],
    )(x_bf, wi_t, wj_t, b_pad)


def _edge1_body(ea_ref, w_ref, e_ref):
    e_ref[...] = _dot(ea_ref[...], w_ref[...])


def _edge1(ea_bf, w_t_bf):
    blk = 512
    nblk_in = N_EDGES // blk
    return pl.pallas_call(
        _edge1_body,
        grid=(E_PAD // blk,),
        in_specs=[
            pl.BlockSpec((blk, DP), lambda i: (jnp.minimum(i, nblk_in - 1), 0)),
            pl.BlockSpec((DP, DP), lambda i: (0, 0)),
        ],
        out_specs=pl.BlockSpec((blk, DP), lambda i: (i, 0)),
        out_shape=jax.ShapeDtypeStruct((E_PAD, DP), _f32),
    )(ea_bf, w_t_bf)


def _post1_body(agg_ref, x_ref, mw1_ref, mb1_ref, mw2_ref, mb2_ref,
                wi2_ref, wj2_ref, b2_ref, x1_ref, xi2_ref, xj2_ref):
    h = (agg_ref[...] + x_ref[...]).astype(_bf16)
    t = jax.nn.relu(_dot(h, mw1_ref[...]) + mb1_ref[0:1, :])
    x1 = jax.nn.relu(_dot(t.astype(_bf16), mw2_ref[...]) + mb2_ref[0:1, :])
    x1_ref[...] = x1
    x1b = x1.astype(_bf16)
    xi2_ref[...] = _dot(x1b, wi2_ref[...]) + b2_ref[0:1, :]
    xj2_ref[...] = _dot(x1b, wj2_ref[...])


def _post1(agg_all, x_all, mw1_t, mb1, mw2_t, mb2, wi2_t, wj2_t, b2):
    n = x_all.shape[0]
    blk = 256
    wspec = pl.BlockSpec((DP, DP), lambda i: (0, 0))
    bspec = pl.BlockSpec((8, DP), lambda i: (0, 0))
    rspec = pl.BlockSpec((blk, DP), lambda i: (i, 0))
    return pl.pallas_call(
        _post1_body,
        grid=(n // blk,),
        in_specs=[rspec, rspec, wspec, bspec, wspec, bspec, wspec, wspec,
                  bspec],
        out_specs=[rspec, rspec, rspec],
        out_shape=[
            jax.ShapeDtypeStruct((n, DP), _f32),
            jax.ShapeDtypeStruct((n, DP), _f32),
            jax.ShapeDtypeStruct((n, DP), _f32),
        ],
    )(agg_all, x_all, mw1_t, mb1, mw2_t, mb2, wi2_t, wj2_t, b2)


def _post2_body(agg_ref, x1_ref, mw1_ref, mb1_ref, mw2_ref, mb2_ref,
                out_ref):
    i = pl.program_id(0)
    blk = agg_ref.shape[0]
    blocks_per_graph = NODE_PAD // blk
    h = (agg_ref[...] + x1_ref[...]).astype(_bf16)
    t = jax.nn.relu(_dot(h, mw1_ref[...]) + mb1_ref[0:1, :])
    x2 = jax.nn.relu(_dot(t.astype(_bf16), mw2_ref[...]) + mb2_ref[0:1, :])
    local_row = (i % blocks_per_graph) * blk + jax.lax.broadcasted_iota(
        jnp.int32, (blk, 1), 0)
    x2 = jnp.where(local_row < N_NODES, x2, 0.0)
    part = x2.reshape(blk // 8, 8, DP).sum(axis=0)

    @pl.when(i % blocks_per_graph == 0)
    def _():
        out_ref[...] = jnp.zeros_like(out_ref)

    out_ref[...] += part


def _post2(agg_all, x1_all, mw1_t, mb1, mw2_t, mb2):
    n = x1_all.shape[0]
    blk = 256
    wspec = pl.BlockSpec((DP, DP), lambda i: (0, 0))
    bspec = pl.BlockSpec((8, DP), lambda i: (0, 0))
    rspec = pl.BlockSpec((blk, DP), lambda i: (i, 0))
    blocks_per_graph = NODE_PAD // blk
    return pl.pallas_call(
        _post2_body,
        grid=(n // blk,),
        in_specs=[rspec, rspec, wspec, bspec, wspec, bspec],
        out_specs=pl.BlockSpec((8, DP), lambda i: (i // blocks_per_graph, 0)),
        out_shape=jax.ShapeDtypeStruct((16, DP), _f32),
    )(agg_all, x1_all, mw1_t, mb1, mw2_t, mb2)


def _sc_conv(xi_all, xj_all, e0, e1, idx_il):
    mesh = plsc.VectorSubcoreMesh(core_axis_name="c", subcore_axis_name="s")
    nct = CHUNKS_PER_TILE

    @functools.partial(
        pl.kernel, mesh=mesh,
        out_type=jax.ShapeDtypeStruct((2 * NODE_PAD, DP), _f32),
        scratch_types=[
            pltpu.VMEM_SHARED((SPMEM_ROWS, DP), _f32),
            pltpu.VMEM((3, 3, CHUNK), jnp.int32),
            pltpu.VMEM((2, CHUNK, DP), _f32),
            pltpu.VMEM((2, CHUNK, DP), _f32),
            pltpu.VMEM((2, CHUNK, DP), _f32),
        ] + [pltpu.SemaphoreType.DMA] * 11,
    )
    def k(xi_h, xj_h, e0_h, e1_h, idx_h, out_h,
          agg_sh, idxb, ri, rj, re,
          sx0, sx1, sx2, si0, si1, sj0, sj1, se0, se1, ss0, ss1):
        sem_idx = [sx0, sx1, sx2]
        sem_i = [si0, si1]
        sem_j = [sj0, sj1]
        sem_e = [se0, se1]
        sem_sc = [ss0, ss1]
        c = jax.lax.axis_index("c")
        s = jax.lax.axis_index("s")
        row0 = s * ROWS_PER_TILE
        cbase = (c * N_SUBCORES + s) * nct
        lbase0 = s * (nct * CHUNK)

        def idx_fetch(kk, islot):
            pltpu.async_copy(idx_h.at[cbase + kk], idxb.at[islot],
                             sem_idx[islot])

        def idx_wait(islot):
            pltpu.make_async_copy(idx_h.at[0], idxb.at[islot],
                                  sem_idx[islot]).wait()

        def gav_start(kk, b, islot):
            lb = lbase0 + kk * CHUNK

            @pl.when(c == 0)
            def _():
                pltpu.async_copy(e0_h.at[pl.ds(lb, CHUNK)], re.at[b],
                                 sem_e[b])

            @pl.when(c == 1)
            def _():
                pltpu.async_copy(e1_h.at[pl.ds(lb, CHUNK)], re.at[b],
                                 sem_e[b])

            pltpu.async_copy(xi_h.at[idxb.at[islot, 1]], ri.at[b], sem_i[b])
            pltpu.async_copy(xj_h.at[idxb.at[islot, 0]], rj.at[b], sem_j[b])

        def gav_wait(b, islot):
            pltpu.make_async_copy(e0_h.at[pl.ds(0, CHUNK)], re.at[b],
                                  sem_e[b]).wait()
            pltpu.make_async_copy(xi_h.at[idxb.at[islot, 1]], ri.at[b],
                                  sem_i[b]).wait()
            pltpu.make_async_copy(xj_h.at[idxb.at[islot, 0]], rj.at[b],
                                  sem_j[b]).wait()

        def scat_start(b, islot):
            pltpu.async_copy(re.at[b], agg_sh.at[idxb.at[islot, 2]],
                             sem_sc[b], add=True)

        def scat_wait(b):
            pltpu.make_async_copy(re.at[b], agg_sh.at[pl.ds(0, CHUNK)],
                                  sem_sc[b]).wait()

        def compute(b):
            def row(i, _):
                for j in range(DP // 16):
                    sl = pl.ds(j * 16, 16)
                    v = re[b, i, sl] + ri[b, i, sl] + rj[b, i, sl]
                    re[b, i, sl] = jnp.maximum(v, 0.0)
                return 0
            jax.lax.fori_loop(0, CHUNK, row, 0)

        idx_fetch(0, 0)
        idx_fetch(1, 1)

        def zrow(i, _):
            for j in range(DP // 16):
                re[0, i, pl.ds(j * 16, 16)] = jnp.zeros((16,), _f32)
            return 0
        jax.lax.fori_loop(0, CHUNK, zrow, 0)
        for off, sz in _COPY_CHUNKS:
            pltpu.sync_copy(re.at[0, pl.ds(0, sz)],
                            agg_sh.at[pl.ds(row0 + off, sz)])
        plsc.subcore_barrier()

        idx_wait(0)
        gav_start(0, 0, 0)

        T = nct // 6

        def body6(t, _):
            k0 = t * 6
            for u in range(6):
                kk = k0 + u
                b, o = u % 2, 1 - u % 2
                icur, inxt, ipre = u % 3, (u + 1) % 3, (u + 2) % 3
                gav_wait(b, icur)

                def launch():
                    idx_wait(inxt)
                    if u == 0:
                        @pl.when(t > 0)
                        def _():
                            scat_wait(o)
                    else:
                        scat_wait(o)
                    gav_start(kk + 1, o, inxt)
                if u < 5:
                    launch()
                else:
                    @pl.when(t < T - 1)
                    def _():
                        launch()

                if u < 4:
                    idx_fetch(kk + 2, ipre)
                else:
                    @pl.when(t < T - 1)
                    def _():
                        idx_fetch(kk + 2, ipre)

                compute(b)
                scat_start(b, icur)
            return 0

        jax.lax.fori_loop(0, T, body6, 0)
        scat_wait(0)
        scat_wait(1)
        plsc.subcore_barrier()

        obase = c * NODE_PAD + row0
        for off, sz in _COPY_CHUNKS:
            pltpu.sync_copy(agg_sh.at[pl.ds(row0 + off, sz)],
                            re.at[0, pl.ds(0, sz)])
            pltpu.sync_copy(re.at[0, pl.ds(0, sz)],
                            out_h.at[pl.ds(obase + off, sz)])

    return k(xi_all, xj_all, e0, e1, idx_il)


def _padw(w, r, c):
    return jnp.zeros((r, c), _f32).at[:w.shape[0], :w.shape[1]].set(w)


def _padb(b):
    return jnp.zeros((8, DP), _f32).at[0, :b.shape[0]].set(b)


def _padidx(a, fill):
    return jnp.full((E_PAD,), fill, jnp.int32).at[:N_EDGES].set(a)


def kernel(node_features_0, node_features_1, edge_features_0, edge_features_1,
           lin1_W, lin1_b, mlp_W1, mlp_b1, mlp_W2, mlp_b2,
           lin2_W, lin2_b, mlp2_W1, mlp2_b1, mlp2_W2, mlp2_b2,
           ntn_W, ntn_V, ntn_b, rule_table, attn_W, gate_W, gate_b,
           fc1_W, fc1_b, fc2_W, fc2_b, fc3_W, fc3_b,
           edge_indices_0, edge_indices_1, rules, ori_lengths):
    x_all = (jnp.zeros((2 * NODE_PAD, DP), _f32)
             .at[:N_NODES, :D].set(node_features_0)
             .at[NODE_PAD:NODE_PAD + N_NODES, :D].set(node_features_1))
    x_bf = x_all.astype(_bf16)
    ea_bf0 = jnp.pad(edge_features_0, ((0, 0), (0, DP - D))).astype(_bf16)
    ea_bf1 = jnp.pad(edge_features_1, ((0, 0), (0, DP - D))).astype(_bf16)

    wi1_t = _padw(lin1_W[:, 0:D].T, DP, DP).astype(_bf16)
    we1_t = _padw(lin1_W[:, D:2 * D].T, DP, DP).astype(_bf16)
    wj1_t = _padw(lin1_W[:, 2 * D:3 * D].T, DP, DP).astype(_bf16)
    b1 = _padb(lin1_b)
    wi2_t = _padw(lin2_W[:, 0:D].T, DP, DP).astype(_bf16)
    we2_t = _padw(lin2_W[:, D:2 * D].T, DP, DP).astype(_bf16)
    wj2_t = _padw(lin2_W[:, 2 * D:3 * D].T, DP, DP).astype(_bf16)
    b2 = _padb(lin2_b)
    mw1_t = _padw(mlp_W1.T, DP, DP).astype(_bf16)
    mb1 = _padb(mlp_b1)
    mw2_t = _padw(mlp_W2.T, DP, DP).astype(_bf16)
    mb2 = _padb(mlp_b2)
    m2w1_t = _padw(mlp2_W1.T, DP, DP).astype(_bf16)
    m2b1 = _padb(mlp2_b1)
    m2w2_t = _padw(mlp2_W2.T, DP, DP).astype(_bf16)
    m2b2 = _padb(mlp2_b2)

    src0, dst0 = edge_indices_0[0], edge_indices_0[1]
    src1, dst1 = edge_indices_1[0], edge_indices_1[1]
    sdx = jnp.concatenate([_padidx(src0, DUMMY),
                           _padidx(src1, DUMMY) + NODE_PAD])
    ddx = jnp.concatenate([_padidx(dst0, DUMMY),
                           _padidx(dst1, DUMMY) + NODE_PAD])
    ddr = jnp.concatenate([_padidx(dst0, SCAT_DUMMY),
                           _padidx(dst1, SCAT_DUMMY)])
    shp = (2, N_SUBCORES, CHUNKS_PER_TILE, CHUNK)
    idx_il = jnp.stack(
        [sdx.reshape(shp), ddx.reshape(shp), ddr.reshape(shp)],
        axis=3).reshape(2 * N_SUBCORES * CHUNKS_PER_TILE, 3, CHUNK)

    xi1_all, xj1_all = _proj(x_bf, wi1_t, wj1_t, b1)
    e1_0 = _edge1(ea_bf0, we1_t)
    e1_1 = _edge1(ea_bf1, we1_t)
    agg1 = _sc_conv(xi1_all, xj1_all, e1_0, e1_1, idx_il)
    e2_0 = _edge1(ea_bf0, we2_t)
    e2_1 = _edge1(ea_bf1, we2_t)
    x1_all, xi2_all, xj2_all = _post1(
        agg1, x_all, mw1_t, mb1, mw2_t, mb2, wi2_t, wj2_t, b2)

    agg2 = _sc_conv(xi2_all, xj2_all, e2_0, e2_1, idx_il)
    colsum = _post2(agg2, x1_all, m2w1_t, m2b1, m2w2_t, m2b2)
    g1 = colsum[0:8].sum(axis=0)[:DIM]
    g2 = colsum[8:16].sum(axis=0)[:DIM]

    bil = jnp.einsum('i,kij,j->k', g1, ntn_W, g2)
    graph_vector = jnp.tanh(bil + ntn_V @ jnp.concatenate([g1, g2]) + ntn_b)
    rule_len = rules.shape[1]
    emb = rule_table[rules]
    mask = (jnp.arange(rule_len)[None, :] < ori_lengths[:, None]).astype(_f32)
    denom = jnp.maximum(ori_lengths, 1).astype(_f32)[:, None]
    rules_embedding = (emb * mask[..., None]).sum(axis=1) / denom
    scores = rules_embedding @ (attn_W @ graph_vector)
    attention_weight = jax.nn.softmax(scores)
    rules_fusion = attention_weight @ rules_embedding
    gate = jax.nn.sigmoid(
        gate_W @ jnp.concatenate([graph_vector, rules_fusion]) + gate_b)
    final_vector = gate * graph_vector + (1.0 - gate) * rules_fusion
    x = jax.nn.relu(fc1_W @ final_vector + fc1_b)
    x = jax.nn.relu(fc2_W @ x + fc2_b)
    x = fc3_W @ x + fc3_b
    return (jnp.abs(x), attention_weight)


# speedup vs baseline: 1.0655x; 1.0655x over previous
import functools

import jax
import jax.numpy as jnp
from jax.experimental import pallas as pl
from jax.experimental.pallas import tpu as pltpu
from jax.experimental.pallas import tpu_sc as plsc

N_NODES = 10000
N_EDGES = 320000
D = 100
DIM = 64

DP = 128
NODE_PAD = 10240
DUMMY = NODE_PAD - 1
N_SUBCORES = 16
N_TILES = 32
CHUNK = 56
CHUNKS_PER_TILE = 180
E_PAD = N_TILES * CHUNK * CHUNKS_PER_TILE
SPMEM_ROWS = 10112
SCAT_DUMMY = N_NODES
ROWS_PER_TILE = SPMEM_ROWS // N_SUBCORES
_COPY_CHUNKS = [(i * CHUNK, CHUNK) for i in range(11)] + [(616, 16)]

_f32 = jnp.float32
_bf16 = jnp.bfloat16


def _dot(a, b):
    return jax.lax.dot_general(a, b, (((1,), (0,)), ((), ())),
                               preferred_element_type=_f32)


def _proj_body(x_ref, wi_ref, wj_ref, b_ref, xi_ref, xj_ref):
    x = x_ref[...]
    xi_ref[...] = _dot(x, wi_ref[...]) + b_ref[0:1, :]
    xj_ref[...] = _dot(x, wj_ref[...])


def _proj(x_bf, wi_t, wj_t, b_pad):
    n = x_bf.shape[0]
    blk = 256
    return pl.pallas_call(
        _proj_body,
        grid=(n // blk,),
        in_specs=[
            pl.BlockSpec((blk, DP), lambda i: (i, 0)),
            pl.BlockSpec((DP, DP), lambda i: (0, 0)),
            pl.BlockSpec((DP, DP), lambda i: (0, 0)),
            pl.BlockSpec((8, DP), lambda i: (0, 0)),
        ],
        out_specs=[
            pl.BlockSpec((blk, DP), lambda i: (i, 0)),
            pl.BlockSpec((blk, DP), lambda i: (i, 0)),
        ],
        out_shape=[
            jax.ShapeDtypeStruct((n, DP), _f32),
            jax.ShapeDtypeStruct((n, DP), _f32),
        ],
    )(x_bf, wi_t, wj_t, b_pad)


def _edge1_body(ea_ref, w_ref, e_ref):
    e_ref[...] = _dot(ea_ref[...], w_ref[...])


def _edge1(ea_bf, w_t_bf):
    blk = 512
    nblk_in = N_EDGES // blk
    return pl.pallas_call(
        _edge1_body,
        grid=(E_PAD // blk,),
        in_specs=[
            pl.BlockSpec((blk, DP), lambda i: (jnp.minimum(i, nblk_in - 1), 0)),
            pl.BlockSpec((DP, DP), lambda i: (0, 0)),
        ],
        out_specs=pl.BlockSpec((blk, DP), lambda i: (i, 0)),
        out_shape=jax.ShapeDtypeStruct((E_PAD, DP), _f32),
    )(ea_bf, w_t_bf)


def _post1_body(agg_ref, aggb_ref, x_ref, mw1_ref, mb1_ref, mw2_ref, mb2_ref,
                wi2_ref, wj2_ref, b2_ref, x1_ref, xi2_ref, xj2_ref):
    h = (agg_ref[...] + aggb_ref[...] + x_ref[...]).astype(_bf16)
    t = jax.nn.relu(_dot(h, mw1_ref[...]) + mb1_ref[0:1, :])
    x1 = jax.nn.relu(_dot(t.astype(_bf16), mw2_ref[...]) + mb2_ref[0:1, :])
    x1_ref[...] = x1
    x1b = x1.astype(_bf16)
    xi2_ref[...] = _dot(x1b, wi2_ref[...]) + b2_ref[0:1, :]
    xj2_ref[...] = _dot(x1b, wj2_ref[...])


def _post1(agg_pair, x_g, mw1_t, mb1, mw2_t, mb2, wi2_t, wj2_t, b2):
    n = x_g.shape[0]
    blk = 256
    nb = NODE_PAD // blk
    wspec = pl.BlockSpec((DP, DP), lambda i: (0, 0))
    bspec = pl.BlockSpec((8, DP), lambda i: (0, 0))
    rspec = pl.BlockSpec((blk, DP), lambda i: (i, 0))
    pspec = pl.BlockSpec((blk, DP), lambda i: (nb + i, 0))
    return pl.pallas_call(
        _post1_body,
        grid=(n // blk,),
        in_specs=[rspec, pspec, rspec, wspec, bspec, wspec, bspec, wspec,
                  wspec, bspec],
        out_specs=[rspec, rspec, rspec],
        out_shape=[
            jax.ShapeDtypeStruct((n, DP), _f32),
            jax.ShapeDtypeStruct((n, DP), _f32),
            jax.ShapeDtypeStruct((n, DP), _f32),
        ],
    )(agg_pair, agg_pair, x_g, mw1_t, mb1, mw2_t, mb2, wi2_t, wj2_t, b2)


def _post2_body(agg_ref, aggb_ref, x1_ref, mw1_ref, mb1_ref, mw2_ref,
                mb2_ref, out_ref):
    i = pl.program_id(0)
    blk = agg_ref.shape[0]
    h = (agg_ref[...] + aggb_ref[...] + x1_ref[...]).astype(_bf16)
    t = jax.nn.relu(_dot(h, mw1_ref[...]) + mb1_ref[0:1, :])
    x2 = jax.nn.relu(_dot(t.astype(_bf16), mw2_ref[...]) + mb2_ref[0:1, :])
    local_row = i * blk + jax.lax.broadcasted_iota(jnp.int32, (blk, 1), 0)
    x2 = jnp.where(local_row < N_NODES, x2, 0.0)
    part = x2.reshape(blk // 8, 8, DP).sum(axis=0)

    @pl.when(i == 0)
    def _():
        out_ref[...] = jnp.zeros_like(out_ref)

    out_ref[...] += part


def _post2(agg_pair, x1_g, mw1_t, mb1, mw2_t, mb2):
    n = x1_g.shape[0]
    blk = 256
    nb = NODE_PAD // blk
    wspec = pl.BlockSpec((DP, DP), lambda i: (0, 0))
    bspec = pl.BlockSpec((8, DP), lambda i: (0, 0))
    rspec = pl.BlockSpec((blk, DP), lambda i: (i, 0))
    pspec = pl.BlockSpec((blk, DP), lambda i: (nb + i, 0))
    return pl.pallas_call(
        _post2_body,
        grid=(n // blk,),
        in_specs=[rspec, pspec, rspec, wspec, bspec, wspec, bspec],
        out_specs=pl.BlockSpec((8, DP), lambda i: (0, 0)),
        out_shape=jax.ShapeDtypeStruct((8, DP), _f32),
    )(agg_pair, agg_pair, x1_g, mw1_t, mb1, mw2_t, mb2)


def _sc_conv(xi_g, xj_g, e_g, idx_il):
    mesh = plsc.VectorSubcoreMesh(core_axis_name="c", subcore_axis_name="s")
    nct = CHUNKS_PER_TILE

    @functools.partial(
        pl.kernel, mesh=mesh,
        out_type=jax.ShapeDtypeStruct((2 * NODE_PAD, DP), _f32),
        scratch_types=[
            pltpu.VMEM_SHARED((SPMEM_ROWS, DP), _f32),
            pltpu.VMEM((3, 3, CHUNK), jnp.int32),
            pltpu.VMEM((2, CHUNK, DP), _f32),
            pltpu.VMEM((2, CHUNK, DP), _f32),
            pltpu.VMEM((2, CHUNK, DP), _f32),
        ] + [pltpu.SemaphoreType.DMA] * 11,
    )
    def k(xi_h, xj_h, e_h, idx_h, out_h,
          agg_sh, idxb, ri, rj, re,
          sx0, sx1, sx2, si0, si1, sj0, sj1, se0, se1, ss0, ss1):
        sem_idx = [sx0, sx1, sx2]
        sem_i = [si0, si1]
        sem_j = [sj0, sj1]
        sem_e = [se0, se1]
        sem_sc = [ss0, ss1]
        c = jax.lax.axis_index("c")
        s = jax.lax.axis_index("s")
        row0 = s * ROWS_PER_TILE
        wid = c * N_SUBCORES + s
        cbase = wid * nct
        lbase0 = wid * (nct * CHUNK)

        def idx_fetch(kk, islot):
            pltpu.async_copy(idx_h.at[cbase + kk], idxb.at[islot],
                             sem_idx[islot])

        def idx_wait(islot):
            pltpu.make_async_copy(idx_h.at[0], idxb.at[islot],
                                  sem_idx[islot]).wait()

        def gav_start(kk, b, islot):
            lb = lbase0 + kk * CHUNK
            pltpu.async_copy(e_h.at[pl.ds(lb, CHUNK)], re.at[b], sem_e[b])
            pltpu.async_copy(xi_h.at[idxb.at[islot, 1]], ri.at[b], sem_i[b])
            pltpu.async_copy(xj_h.at[idxb.at[islot, 0]], rj.at[b], sem_j[b])

        def gav_wait(b, islot):
            pltpu.make_async_copy(e_h.at[pl.ds(0, CHUNK)], re.at[b],
                                  sem_e[b]).wait()
            pltpu.make_async_copy(xi_h.at[idxb.at[islot, 1]], ri.at[b],
                                  sem_i[b]).wait()
            pltpu.make_async_copy(xj_h.at[idxb.at[islot, 0]], rj.at[b],
                                  sem_j[b]).wait()

        def scat_start(b, islot):
            pltpu.async_copy(re.at[b], agg_sh.at[idxb.at[islot, 2]],
                             sem_sc[b], add=True)

        def scat_wait(b):
            pltpu.make_async_copy(re.at[b], agg_sh.at[pl.ds(0, CHUNK)],
                                  sem_sc[b]).wait()

        def compute(b):
            def row(i, _):
                for j in range(DP // 16):
                    sl = pl.ds(j * 16, 16)
                    v = re[b, i, sl] + ri[b, i, sl] + rj[b, i, sl]
                    re[b, i, sl] = jnp.maximum(v, 0.0)
                return 0
            jax.lax.fori_loop(0, CHUNK, row, 0)

        idx_fetch(0, 0)
        idx_fetch(1, 1)

        def zrow(i, _):
            for j in range(DP // 16):
                re[0, i, pl.ds(j * 16, 16)] = jnp.zeros((16,), _f32)
            return 0
        jax.lax.fori_loop(0, CHUNK, zrow, 0)
        for off, sz in _COPY_CHUNKS:
            pltpu.sync_copy(re.at[0, pl.ds(0, sz)],
                            agg_sh.at[pl.ds(row0 + off, sz)])
        plsc.subcore_barrier()

        idx_wait(0)
        gav_start(0, 0, 0)

        T = nct // 6

        def body6(t, _):
            k0 = t * 6
            for u in range(6):
                kk = k0 + u
                b, o = u % 2, 1 - u % 2
                icur, inxt, ipre = u % 3, (u + 1) % 3, (u + 2) % 3
                gav_wait(b, icur)

                def launch():
                    idx_wait(inxt)
                    if u == 0:
                        @pl.when(t > 0)
                        def _():
                            scat_wait(o)
                    else:
                        scat_wait(o)
                    gav_start(kk + 1, o, inxt)
                if u < 5:
                    launch()
                else:
                    @pl.when(t < T - 1)
                    def _():
                        launch()

                if u < 4:
                    idx_fetch(kk + 2, ipre)
                else:
                    @pl.when(t < T - 1)
                    def _():
                        idx_fetch(kk + 2, ipre)

                compute(b)
                scat_start(b, icur)
            return 0

        jax.lax.fori_loop(0, T, body6, 0)
        scat_wait(0)
        scat_wait(1)
        plsc.subcore_barrier()

        obase = c * NODE_PAD + row0
        for off, sz in _COPY_CHUNKS:
            pltpu.sync_copy(agg_sh.at[pl.ds(row0 + off, sz)],
                            re.at[0, pl.ds(0, sz)])
            pltpu.sync_copy(re.at[0, pl.ds(0, sz)],
                            out_h.at[pl.ds(obase + off, sz)])

    return k(xi_g, xj_g, e_g, idx_il)


def _padw(w, r, c):
    return jnp.zeros((r, c), _f32).at[:w.shape[0], :w.shape[1]].set(w)


def _padb(b):
    return jnp.zeros((8, DP), _f32).at[0, :b.shape[0]].set(b)


def _padidx(a, fill):
    return jnp.full((E_PAD,), fill, jnp.int32).at[:N_EDGES].set(a)


def kernel(node_features_0, node_features_1, edge_features_0, edge_features_1,
           lin1_W, lin1_b, mlp_W1, mlp_b1, mlp_W2, mlp_b2,
           lin2_W, lin2_b, mlp2_W1, mlp2_b1, mlp2_W2, mlp2_b2,
           ntn_W, ntn_V, ntn_b, rule_table, attn_W, gate_W, gate_b,
           fc1_W, fc1_b, fc2_W, fc2_b, fc3_W, fc3_b,
           edge_indices_0, edge_indices_1, rules, ori_lengths):
    x_g0 = jnp.zeros((NODE_PAD, DP), _f32).at[:N_NODES, :D].set(
        node_features_0)
    x_g1 = jnp.zeros((NODE_PAD, DP), _f32).at[:N_NODES, :D].set(
        node_features_1)
    ea_bf0 = jnp.pad(edge_features_0, ((0, 0), (0, DP - D))).astype(_bf16)
    ea_bf1 = jnp.pad(edge_features_1, ((0, 0), (0, DP - D))).astype(_bf16)

    wi1_t = _padw(lin1_W[:, 0:D].T, DP, DP).astype(_bf16)
    we1_t = _padw(lin1_W[:, D:2 * D].T, DP, DP).astype(_bf16)
    wj1_t = _padw(lin1_W[:, 2 * D:3 * D].T, DP, DP).astype(_bf16)
    b1 = _padb(lin1_b)
    wi2_t = _padw(lin2_W[:, 0:D].T, DP, DP).astype(_bf16)
    we2_t = _padw(lin2_W[:, D:2 * D].T, DP, DP).astype(_bf16)
    wj2_t = _padw(lin2_W[:, 2 * D:3 * D].T, DP, DP).astype(_bf16)
    b2 = _padb(lin2_b)
    mw1_t = _padw(mlp_W1.T, DP, DP).astype(_bf16)
    mb1 = _padb(mlp_b1)
    mw2_t = _padw(mlp_W2.T, DP, DP).astype(_bf16)
    mb2 = _padb(mlp_b2)
    m2w1_t = _padw(mlp2_W1.T, DP, DP).astype(_bf16)
    m2b1 = _padb(mlp2_b1)
    m2w2_t = _padw(mlp2_W2.T, DP, DP).astype(_bf16)
    m2b2 = _padb(mlp2_b2)

    def mk_idx(ei):
        src, dst = ei[0], ei[1]
        shp = (N_TILES, CHUNKS_PER_TILE, CHUNK)
        return jnp.stack(
            [_padidx(src, DUMMY).reshape(shp),
             _padidx(dst, DUMMY).reshape(shp),
             _padidx(dst, SCAT_DUMMY).reshape(shp)],
            axis=2).reshape(N_TILES * CHUNKS_PER_TILE, 3, CHUNK)

    idx_g0 = mk_idx(edge_indices_0)
    idx_g1 = mk_idx(edge_indices_1)

    xi1_g0, xj1_g0 = _proj(x_g0.astype(_bf16), wi1_t, wj1_t, b1)
    e1_0 = _edge1(ea_bf0, we1_t)
    agg1_g0 = _sc_conv(xi1_g0, xj1_g0, e1_0, idx_g0)

    xi1_g1, xj1_g1 = _proj(x_g1.astype(_bf16), wi1_t, wj1_t, b1)
    e1_1 = _edge1(ea_bf1, we1_t)
    e2_0 = _edge1(ea_bf0, we2_t)
    agg1_g1 = _sc_conv(xi1_g1, xj1_g1, e1_1, idx_g1)

    x1_g0, xi2_g0, xj2_g0 = _post1(
        agg1_g0, x_g0, mw1_t, mb1, mw2_t, mb2, wi2_t, wj2_t, b2)
    agg2_g0 = _sc_conv(xi2_g0, xj2_g0, e2_0, idx_g0)

    e2_1 = _edge1(ea_bf1, we2_t)
    x1_g1, xi2_g1, xj2_g1 = _post1(
        agg1_g1, x_g1, mw1_t, mb1, mw2_t, mb2, wi2_t, wj2_t, b2)
    agg2_g1 = _sc_conv(xi2_g1, xj2_g1, e2_1, idx_g1)

    cs0 = _post2(agg2_g0, x1_g0, m2w1_t, m2b1, m2w2_t, m2b2)
    cs1 = _post2(agg2_g1, x1_g1, m2w1_t, m2b1, m2w2_t, m2b2)
    g1 = cs0.sum(axis=0)[:DIM]
    g2 = cs1.sum(axis=0)[:DIM]

    bil = jnp.einsum('i,kij,j->k', g1, ntn_W, g2)
    graph_vector = jnp.tanh(bil + ntn_V @ jnp.concatenate([g1, g2]) + ntn_b)
    rule_len = rules.shape[1]
    emb = rule_table[rules]
    mask = (jnp.arange(rule_len)[None, :] < ori_lengths[:, None]).astype(_f32)
    denom = jnp.maximum(ori_lengths, 1).astype(_f32)[:, None]
    rules_embedding = (emb * mask[..., None]).sum(axis=1) / denom
    scores = rules_embedding @ (attn_W @ graph_vector)
    attention_weight = jax.nn.softmax(scores)
    rules_fusion = attention_weight @ rules_embedding
    gate = jax.nn.sigmoid(
        gate_W @ jnp.concatenate([graph_vector, rules_fusion]) + gate_b)
    final_vector = gate * graph_vector + (1.0 - gate) * rules_fusion
    x = jax.nn.relu(fc1_W @ final_vector + fc1_b)
    x = jax.nn.relu(fc2_W @ x + fc2_b)
    x = fc3_W @ x + fc3_b
    return (jnp.abs(x), attention_weight)


# speedup vs baseline: 1.2873x; 1.2081x over previous
import functools

import jax
import jax.numpy as jnp
from jax.experimental import pallas as pl
from jax.experimental.pallas import tpu as pltpu
from jax.experimental.pallas import tpu_sc as plsc

N_NODES = 10000
N_EDGES = 320000
D = 100
DIM = 64

DP = 128
NODE_PAD = 10240
DUMMY = NODE_PAD - 1
N_SUBCORES = 16
N_TILES = 32
CHUNK = 56
CHUNKS_PER_TILE = 180
NCT0 = 210
NCT1 = 150
E_PAD = N_TILES * CHUNK * CHUNKS_PER_TILE
SPMEM_ROWS = 10112
SCAT_DUMMY = N_NODES
ROWS_PER_TILE = SPMEM_ROWS // N_SUBCORES
_COPY_CHUNKS = [(i * CHUNK, CHUNK) for i in range(11)] + [(616, 16)]

_f32 = jnp.float32
_bf16 = jnp.bfloat16


def _dot(a, b):
    return jax.lax.dot_general(a, b, (((1,), (0,)), ((), ())),
                               preferred_element_type=_f32)


def _proj_body(x_ref, wi_ref, wj_ref, b_ref, xi_ref, xj_ref):
    x = x_ref[...]
    xi_ref[...] = _dot(x, wi_ref[...]) + b_ref[0:1, :]
    xj_ref[...] = _dot(x, wj_ref[...])


def _proj(x_bf, wi_t, wj_t, b_pad):
    n = x_bf.shape[0]
    blk = 256
    return pl.pallas_call(
        _proj_body,
        grid=(n // blk,),
        in_specs=[
            pl.BlockSpec((blk, DP), lambda i: (i, 0)),
            pl.BlockSpec((DP, DP), lambda i: (0, 0)),
            pl.BlockSpec((DP, DP), lambda i: (0, 0)),
            pl.BlockSpec((8, DP), lambda i: (0, 0)),
        ],
        out_specs=[
            pl.BlockSpec((blk, DP), lambda i: (i, 0)),
            pl.BlockSpec((blk, DP), lambda i: (i, 0)),
        ],
        out_shape=[
            jax.ShapeDtypeStruct((n, DP), _f32),
            jax.ShapeDtypeStruct((n, DP), _f32),
        ],
    )(x_bf, wi_t, wj_t, b_pad)


def _edge1_body(ea_ref, w_ref, e_ref):
    e_ref[...] = _dot(ea_ref[...], w_ref[...])


def _edge1(ea_bf, w_t_bf):
    blk = 1280
    nblk_in = N_EDGES // blk
    return pl.pallas_call(
        _edge1_body,
        grid=(E_PAD // blk,),
        in_specs=[
            pl.BlockSpec((blk, DP), lambda i: (jnp.minimum(i, nblk_in - 1), 0)),
            pl.BlockSpec((DP, DP), lambda i: (0, 0)),
        ],
        out_specs=pl.BlockSpec((blk, DP), lambda i: (i, 0)),
        out_shape=jax.ShapeDtypeStruct((E_PAD, DP), _f32),
    )(ea_bf, w_t_bf)


def _post1_body(agg_ref, aggb_ref, x_ref, mw1_ref, mb1_ref, mw2_ref, mb2_ref,
                wi2_ref, wj2_ref, b2_ref, x1_ref, xi2_ref, xj2_ref):
    h = (agg_ref[...] + aggb_ref[...] + x_ref[...]).astype(_bf16)
    t = jax.nn.relu(_dot(h, mw1_ref[...]) + mb1_ref[0:1, :])
    x1 = jax.nn.relu(_dot(t.astype(_bf16), mw2_ref[...]) + mb2_ref[0:1, :])
    x1_ref[...] = x1
    x1b = x1.astype(_bf16)
    xi2_ref[...] = _dot(x1b, wi2_ref[...]) + b2_ref[0:1, :]
    xj2_ref[...] = _dot(x1b, wj2_ref[...])


def _post1(agg_pair, x_g, mw1_t, mb1, mw2_t, mb2, wi2_t, wj2_t, b2):
    n = x_g.shape[0]
    blk = 256
    nb = NODE_PAD // blk
    wspec = pl.BlockSpec((DP, DP), lambda i: (0, 0))
    bspec = pl.BlockSpec((8, DP), lambda i: (0, 0))
    rspec = pl.BlockSpec((blk, DP), lambda i: (i, 0))
    pspec = pl.BlockSpec((blk, DP), lambda i: (nb + i, 0))
    return pl.pallas_call(
        _post1_body,
        grid=(n // blk,),
        in_specs=[rspec, pspec, rspec, wspec, bspec, wspec, bspec, wspec,
                  wspec, bspec],
        out_specs=[rspec, rspec, rspec],
        out_shape=[
            jax.ShapeDtypeStruct((n, DP), _f32),
            jax.ShapeDtypeStruct((n, DP), _f32),
            jax.ShapeDtypeStruct((n, DP), _f32),
        ],
    )(agg_pair, agg_pair, x_g, mw1_t, mb1, mw2_t, mb2, wi2_t, wj2_t, b2)


def _post2_body(agg_ref, aggb_ref, x1_ref, mw1_ref, mb1_ref, mw2_ref,
                mb2_ref, out_ref):
    i = pl.program_id(0)
    blk = agg_ref.shape[0]
    h = (agg_ref[...] + aggb_ref[...] + x1_ref[...]).astype(_bf16)
    t = jax.nn.relu(_dot(h, mw1_ref[...]) + mb1_ref[0:1, :])
    x2 = jax.nn.relu(_dot(t.astype(_bf16), mw2_ref[...]) + mb2_ref[0:1, :])
    local_row = i * blk + jax.lax.broadcasted_iota(jnp.int32, (blk, 1), 0)
    x2 = jnp.where(local_row < N_NODES, x2, 0.0)
    part = x2.reshape(blk // 8, 8, DP).sum(axis=0)

    @pl.when(i == 0)
    def _():
        out_ref[...] = jnp.zeros_like(out_ref)

    out_ref[...] += part


def _post2(agg_pair, x1_g, mw1_t, mb1, mw2_t, mb2):
    n = x1_g.shape[0]
    blk = 256
    nb = NODE_PAD // blk
    wspec = pl.BlockSpec((DP, DP), lambda i: (0, 0))
    bspec = pl.BlockSpec((8, DP), lambda i: (0, 0))
    rspec = pl.BlockSpec((blk, DP), lambda i: (i, 0))
    pspec = pl.BlockSpec((blk, DP), lambda i: (nb + i, 0))
    return pl.pallas_call(
        _post2_body,
        grid=(n // blk,),
        in_specs=[rspec, pspec, rspec, wspec, bspec, wspec, bspec],
        out_specs=pl.BlockSpec((8, DP), lambda i: (0, 0)),
        out_shape=jax.ShapeDtypeStruct((8, DP), _f32),
    )(agg_pair, agg_pair, x1_g, mw1_t, mb1, mw2_t, mb2)


def _sc_conv(xi_g, xj_g, e_g, idx_il):
    mesh = plsc.VectorSubcoreMesh(core_axis_name="c", subcore_axis_name="s")
    nct = CHUNKS_PER_TILE

    @functools.partial(
        pl.kernel, mesh=mesh,
        out_type=jax.ShapeDtypeStruct((2 * NODE_PAD, DP), _f32),
        scratch_types=[
            pltpu.VMEM_SHARED((SPMEM_ROWS, DP), _f32),
            pltpu.VMEM((3, 3, CHUNK), jnp.int32),
            pltpu.VMEM((2, CHUNK, DP), _f32),
            pltpu.VMEM((2, CHUNK, DP), _f32),
            pltpu.VMEM((2, CHUNK, DP), _f32),
        ] + [pltpu.SemaphoreType.DMA] * 11,
    )
    def k(xi_h, xj_h, e_h, idx_h, out_h,
          agg_sh, idxb, ri, rj, re,
          sx0, sx1, sx2, si0, si1, sj0, sj1, se0, se1, ss0, ss1):
        sem_idx = [sx0, sx1, sx2]
        sem_i = [si0, si1]
        sem_j = [sj0, sj1]
        sem_e = [se0, se1]
        sem_sc = [ss0, ss1]
        c = jax.lax.axis_index("c")
        s = jax.lax.axis_index("s")
        row0 = s * ROWS_PER_TILE
        cbase = jnp.where(c == 0, s * NCT0, N_SUBCORES * NCT0 + s * NCT1)
        lbase0 = cbase * CHUNK
        T = jnp.where(c == 0, NCT0 // 6, NCT1 // 6)

        def idx_fetch(kk, islot):
            pltpu.async_copy(idx_h.at[cbase + kk], idxb.at[islot],
                             sem_idx[islot])

        def idx_wait(islot):
            pltpu.make_async_copy(idx_h.at[0], idxb.at[islot],
                                  sem_idx[islot]).wait()

        def gav_start(kk, b, islot):
            lb = lbase0 + kk * CHUNK
            pltpu.async_copy(e_h.at[pl.ds(lb, CHUNK)], re.at[b], sem_e[b])
            pltpu.async_copy(xi_h.at[idxb.at[islot, 1]], ri.at[b], sem_i[b])
            pltpu.async_copy(xj_h.at[idxb.at[islot, 0]], rj.at[b], sem_j[b])

        def gav_wait(b, islot):
            pltpu.make_async_copy(e_h.at[pl.ds(0, CHUNK)], re.at[b],
                                  sem_e[b]).wait()
            pltpu.make_async_copy(xi_h.at[idxb.at[islot, 1]], ri.at[b],
                                  sem_i[b]).wait()
            pltpu.make_async_copy(xj_h.at[idxb.at[islot, 0]], rj.at[b],
                                  sem_j[b]).wait()

        def scat_start(b, islot):
            pltpu.async_copy(re.at[b], agg_sh.at[idxb.at[islot, 2]],
                             sem_sc[b], add=True)

        def scat_wait(b):
            pltpu.make_async_copy(re.at[b], agg_sh.at[pl.ds(0, CHUNK)],
                                  sem_sc[b]).wait()

        def compute(b):
            def row(i, _):
                for j in range(DP // 16):
                    sl = pl.ds(j * 16, 16)
                    v = re[b, i, sl] + ri[b, i, sl] + rj[b, i, sl]
                    re[b, i, sl] = jnp.maximum(v, 0.0)
                return 0
            jax.lax.fori_loop(0, CHUNK, row, 0)

        idx_fetch(0, 0)
        idx_fetch(1, 1)

        def zrow(i, _):
            for j in range(DP // 16):
                re[0, i, pl.ds(j * 16, 16)] = jnp.zeros((16,), _f32)
            return 0
        jax.lax.fori_loop(0, CHUNK, zrow, 0)
        for off, sz in _COPY_CHUNKS:
            pltpu.sync_copy(re.at[0, pl.ds(0, sz)],
                            agg_sh.at[pl.ds(row0 + off, sz)])
        plsc.subcore_barrier()

        idx_wait(0)
        gav_start(0, 0, 0)

        def body6(t, _):
            k0 = t * 6
            for u in range(6):
                kk = k0 + u
                b, o = u % 2, 1 - u % 2
                icur, inxt, ipre = u % 3, (u + 1) % 3, (u + 2) % 3
                gav_wait(b, icur)

                def launch():
                    idx_wait(inxt)
                    if u == 0:
                        @pl.when(t > 0)
                        def _():
                            scat_wait(o)
                    else:
                        scat_wait(o)
                    gav_start(kk + 1, o, inxt)
                if u < 5:
                    launch()
                else:
                    @pl.when(t < T - 1)
                    def _():
                        launch()

                if u < 4:
                    idx_fetch(kk + 2, ipre)
                else:
                    @pl.when(t < T - 1)
                    def _():
                        idx_fetch(kk + 2, ipre)

                compute(b)
                scat_start(b, icur)
            return 0

        jax.lax.fori_loop(0, T, body6, 0)
        scat_wait(0)
        scat_wait(1)
        plsc.subcore_barrier()

        obase = c * NODE_PAD + row0
        for off, sz in _COPY_CHUNKS:
            pltpu.sync_copy(agg_sh.at[pl.ds(row0 + off, sz)],
                            re.at[0, pl.ds(0, sz)])
            pltpu.sync_copy(re.at[0, pl.ds(0, sz)],
                            out_h.at[pl.ds(obase + off, sz)])

    return k(xi_g, xj_g, e_g, idx_il)


def _padw(w, r, c):
    return jnp.zeros((r, c), _f32).at[:w.shape[0], :w.shape[1]].set(w)


def _padb(b):
    return jnp.zeros((8, DP), _f32).at[0, :b.shape[0]].set(b)


def _padidx(a, fill):
    return jnp.full((E_PAD,), fill, jnp.int32).at[:N_EDGES].set(a)


def kernel(node_features_0, node_features_1, edge_features_0, edge_features_1,
           lin1_W, lin1_b, mlp_W1, mlp_b1, mlp_W2, mlp_b2,
           lin2_W, lin2_b, mlp2_W1, mlp2_b1, mlp2_W2, mlp2_b2,
           ntn_W, ntn_V, ntn_b, rule_table, attn_W, gate_W, gate_b,
           fc1_W, fc1_b, fc2_W, fc2_b, fc3_W, fc3_b,
           edge_indices_0, edge_indices_1, rules, ori_lengths):
    x_g0 = jnp.zeros((NODE_PAD, DP), _f32).at[:N_NODES, :D].set(
        node_features_0)
    x_g1 = jnp.zeros((NODE_PAD, DP), _f32).at[:N_NODES, :D].set(
        node_features_1)
    ea_bf0 = jnp.pad(edge_features_0, ((0, 0), (0, DP - D))).astype(_bf16)
    ea_bf1 = jnp.pad(edge_features_1, ((0, 0), (0, DP - D))).astype(_bf16)

    wi1_t = _padw(lin1_W[:, 0:D].T, DP, DP).astype(_bf16)
    we1_t = _padw(lin1_W[:, D:2 * D].T, DP, DP).astype(_bf16)
    wj1_t = _padw(lin1_W[:, 2 * D:3 * D].T, DP, DP).astype(_bf16)
    b1 = _padb(lin1_b)
    wi2_t = _padw(lin2_W[:, 0:D].T, DP, DP).astype(_bf16)
    we2_t = _padw(lin2_W[:, D:2 * D].T, DP, DP).astype(_bf16)
    wj2_t = _padw(lin2_W[:, 2 * D:3 * D].T, DP, DP).astype(_bf16)
    b2 = _padb(lin2_b)
    mw1_t = _padw(mlp_W1.T, DP, DP).astype(_bf16)
    mb1 = _padb(mlp_b1)
    mw2_t = _padw(mlp_W2.T, DP, DP).astype(_bf16)
    mb2 = _padb(mlp_b2)
    m2w1_t = _padw(mlp2_W1.T, DP, DP).astype(_bf16)
    m2b1 = _padb(mlp2_b1)
    m2w2_t = _padw(mlp2_W2.T, DP, DP).astype(_bf16)
    m2b2 = _padb(mlp2_b2)

    def mk_idx(ei):
        src, dst = ei[0], ei[1]
        shp = (N_TILES, CHUNKS_PER_TILE, CHUNK)
        return jnp.stack(
            [_padidx(src, DUMMY).reshape(shp),
             _padidx(dst, DUMMY).reshape(shp),
             _padidx(dst, SCAT_DUMMY).reshape(shp)],
            axis=2).reshape(N_TILES * CHUNKS_PER_TILE, 3, CHUNK)

    idx_g0 = mk_idx(edge_indices_0)
    idx_g1 = mk_idx(edge_indices_1)

    xi1_g0, xj1_g0 = _proj(x_g0.astype(_bf16), wi1_t, wj1_t, b1)
    e1_0 = _edge1(ea_bf0, we1_t)
    agg1_g0 = _sc_conv(xi1_g0, xj1_g0, e1_0, idx_g0)

    xi1_g1, xj1_g1 = _proj(x_g1.astype(_bf16), wi1_t, wj1_t, b1)
    e1_1 = _edge1(ea_bf1, we1_t)
    e2_0 = _edge1(ea_bf0, we2_t)
    agg1_g1 = _sc_conv(xi1_g1, xj1_g1, e1_1, idx_g1)

    x1_g0, xi2_g0, xj2_g0 = _post1(
        agg1_g0, x_g0, mw1_t, mb1, mw2_t, mb2, wi2_t, wj2_t, b2)
    agg2_g0 = _sc_conv(xi2_g0, xj2_g0, e2_0, idx_g0)

    e2_1 = _edge1(ea_bf1, we2_t)
    x1_g1, xi2_g1, xj2_g1 = _post1(
        agg1_g1, x_g1, mw1_t, mb1, mw2_t, mb2, wi2_t, wj2_t, b2)
    agg2_g1 = _sc_conv(xi2_g1, xj2_g1, e2_1, idx_g1)

    cs0 = _post2(agg2_g0, x1_g0, m2w1_t, m2b1, m2w2_t, m2b2)
    cs1 = _post2(agg2_g1, x1_g1, m2w1_t, m2b1, m2w2_t, m2b2)
    g1 = cs0.sum(axis=0)[:DIM]
    g2 = cs1.sum(axis=0)[:DIM]

    bil = jnp.einsum('i,kij,j->k', g1, ntn_W, g2)
    graph_vector = jnp.tanh(bil + ntn_V @ jnp.concatenate([g1, g2]) + ntn_b)
    rule_len = rules.shape[1]
    emb = rule_table[rules]
    mask = (jnp.arange(rule_len)[None, :] < ori_lengths[:, None]).astype(_f32)
    denom = jnp.maximum(ori_lengths, 1).astype(_f32)[:, None]
    rules_embedding = (emb * mask[..., None]).sum(axis=1) / denom
    scores = rules_embedding @ (attn_W @ graph_vector)
    attention_weight = jax.nn.softmax(scores)
    rules_fusion = attention_weight @ rules_embedding
    gate = jax.nn.sigmoid(
        gate_W @ jnp.concatenate([graph_vector, rules_fusion]) + gate_b)
    final_vector = gate * graph_vector + (1.0 - gate) * rules_fusion
    x = jax.nn.relu(fc1_W @ final_vector + fc1_b)
    x = jax.nn.relu(fc2_W @ x + fc2_b)
    x = fc3_W @ x + fc3_b
    return (jnp.abs(x), attention_weight)


# speedup vs baseline: 1.3056x; 1.0142x over previous
import functools

import jax
import jax.numpy as jnp
from jax.experimental import pallas as pl
from jax.experimental.pallas import tpu as pltpu
from jax.experimental.pallas import tpu_sc as plsc

N_NODES = 10000
N_EDGES = 320000
D = 100
DIM = 64

DP = 128
NODE_PAD = 10240
DUMMY = NODE_PAD - 1
N_SUBCORES = 16
N_TILES = 32
CHUNK = 56
CHUNKS_PER_TILE = 180
NCT0 = 222
NCT1 = 138
E_PAD = N_TILES * CHUNK * CHUNKS_PER_TILE
SPMEM_ROWS = 10112
SCAT_DUMMY = N_NODES
ROWS_PER_TILE = SPMEM_ROWS // N_SUBCORES
_COPY_CHUNKS = [(i * CHUNK, CHUNK) for i in range(11)] + [(616, 16)]

_f32 = jnp.float32
_bf16 = jnp.bfloat16


def _dot(a, b):
    return jax.lax.dot_general(a, b, (((1,), (0,)), ((), ())),
                               preferred_element_type=_f32)


def _proj_body(x_ref, wi_ref, wj_ref, b_ref, xi_ref, xj_ref):
    x = x_ref[...]
    xi_ref[...] = _dot(x, wi_ref[...]) + b_ref[0:1, :]
    xj_ref[...] = _dot(x, wj_ref[...])


def _proj(x_bf, wi_t, wj_t, b_pad):
    n = x_bf.shape[0]
    blk = 256
    return pl.pallas_call(
        _proj_body,
        grid=(n // blk,),
        in_specs=[
            pl.BlockSpec((blk, DP), lambda i: (i, 0)),
            pl.BlockSpec((DP, DP), lambda i: (0, 0)),
            pl.BlockSpec((DP, DP), lambda i: (0, 0)),
            pl.BlockSpec((8, DP), lambda i: (0, 0)),
        ],
        out_specs=[
            pl.BlockSpec((blk, DP), lambda i: (i, 0)),
            pl.BlockSpec((blk, DP), lambda i: (i, 0)),
        ],
        out_shape=[
            jax.ShapeDtypeStruct((n, DP), _f32),
            jax.ShapeDtypeStruct((n, DP), _f32),
        ],
    )(x_bf, wi_t, wj_t, b_pad)


def _edge1_body(ea_ref, w_ref, e_ref):
    e_ref[...] = _dot(ea_ref[...], w_ref[...])


def _edge1(ea_bf, w_t_bf):
    blk = 1280
    nblk_in = N_EDGES // blk
    return pl.pallas_call(
        _edge1_body,
        grid=(E_PAD // blk,),
        in_specs=[
            pl.BlockSpec((blk, DP), lambda i: (jnp.minimum(i, nblk_in - 1), 0)),
            pl.BlockSpec((DP, DP), lambda i: (0, 0)),
        ],
        out_specs=pl.BlockSpec((blk, DP), lambda i: (i, 0)),
        out_shape=jax.ShapeDtypeStruct((E_PAD, DP), _f32),
    )(ea_bf, w_t_bf)


def _post1_body(agg_ref, aggb_ref, x_ref, mw1_ref, mb1_ref, mw2_ref, mb2_ref,
                wi2_ref, wj2_ref, b2_ref, x1_ref, xi2_ref, xj2_ref):
    h = (agg_ref[...] + aggb_ref[...] + x_ref[...]).astype(_bf16)
    t = jax.nn.relu(_dot(h, mw1_ref[...]) + mb1_ref[0:1, :])
    x1 = jax.nn.relu(_dot(t.astype(_bf16), mw2_ref[...]) + mb2_ref[0:1, :])
    x1_ref[...] = x1
    x1b = x1.astype(_bf16)
    xi2_ref[...] = _dot(x1b, wi2_ref[...]) + b2_ref[0:1, :]
    xj2_ref[...] = _dot(x1b, wj2_ref[...])


def _post1(agg_pair, x_g, mw1_t, mb1, mw2_t, mb2, wi2_t, wj2_t, b2):
    n = x_g.shape[0]
    blk = 256
    nb = NODE_PAD // blk
    wspec = pl.BlockSpec((DP, DP), lambda i: (0, 0))
    bspec = pl.BlockSpec((8, DP), lambda i: (0, 0))
    rspec = pl.BlockSpec((blk, DP), lambda i: (i, 0))
    pspec = pl.BlockSpec((blk, DP), lambda i: (nb + i, 0))
    return pl.pallas_call(
        _post1_body,
        grid=(n // blk,),
        in_specs=[rspec, pspec, rspec, wspec, bspec, wspec, bspec, wspec,
                  wspec, bspec],
        out_specs=[rspec, rspec, rspec],
        out_shape=[
            jax.ShapeDtypeStruct((n, DP), _f32),
            jax.ShapeDtypeStruct((n, DP), _f32),
            jax.ShapeDtypeStruct((n, DP), _f32),
        ],
    )(agg_pair, agg_pair, x_g, mw1_t, mb1, mw2_t, mb2, wi2_t, wj2_t, b2)


def _post2_body(agg_ref, aggb_ref, x1_ref, mw1_ref, mb1_ref, mw2_ref,
                mb2_ref, out_ref):
    i = pl.program_id(0)
    blk = agg_ref.shape[0]
    h = (agg_ref[...] + aggb_ref[...] + x1_ref[...]).astype(_bf16)
    t = jax.nn.relu(_dot(h, mw1_ref[...]) + mb1_ref[0:1, :])
    x2 = jax.nn.relu(_dot(t.astype(_bf16), mw2_ref[...]) + mb2_ref[0:1, :])
    local_row = i * blk + jax.lax.broadcasted_iota(jnp.int32, (blk, 1), 0)
    x2 = jnp.where(local_row < N_NODES, x2, 0.0)
    part = x2.reshape(blk // 8, 8, DP).sum(axis=0)

    @pl.when(i == 0)
    def _():
        out_ref[...] = jnp.zeros_like(out_ref)

    out_ref[...] += part


def _post2(agg_pair, x1_g, mw1_t, mb1, mw2_t, mb2):
    n = x1_g.shape[0]
    blk = 256
    nb = NODE_PAD // blk
    wspec = pl.BlockSpec((DP, DP), lambda i: (0, 0))
    bspec = pl.BlockSpec((8, DP), lambda i: (0, 0))
    rspec = pl.BlockSpec((blk, DP), lambda i: (i, 0))
    pspec = pl.BlockSpec((blk, DP), lambda i: (nb + i, 0))
    return pl.pallas_call(
        _post2_body,
        grid=(n // blk,),
        in_specs=[rspec, pspec, rspec, wspec, bspec, wspec, bspec],
        out_specs=pl.BlockSpec((8, DP), lambda i: (0, 0)),
        out_shape=jax.ShapeDtypeStruct((8, DP), _f32),
    )(agg_pair, agg_pair, x1_g, mw1_t, mb1, mw2_t, mb2)


def _sc_conv(xi_g, xj_g, e_g, idx_il):
    mesh = plsc.VectorSubcoreMesh(core_axis_name="c", subcore_axis_name="s")
    nct = CHUNKS_PER_TILE

    @functools.partial(
        pl.kernel, mesh=mesh,
        out_type=jax.ShapeDtypeStruct((2 * NODE_PAD, DP), _f32),
        scratch_types=[
            pltpu.VMEM_SHARED((SPMEM_ROWS, DP), _f32),
            pltpu.VMEM((3, 3, CHUNK), jnp.int32),
            pltpu.VMEM((2, CHUNK, DP), _f32),
            pltpu.VMEM((2, CHUNK, DP), _f32),
            pltpu.VMEM((2, CHUNK, DP), _f32),
        ] + [pltpu.SemaphoreType.DMA] * 11,
    )
    def k(xi_h, xj_h, e_h, idx_h, out_h,
          agg_sh, idxb, ri, rj, re,
          sx0, sx1, sx2, si0, si1, sj0, sj1, se0, se1, ss0, ss1):
        sem_idx = [sx0, sx1, sx2]
        sem_i = [si0, si1]
        sem_j = [sj0, sj1]
        sem_e = [se0, se1]
        sem_sc = [ss0, ss1]
        c = jax.lax.axis_index("c")
        s = jax.lax.axis_index("s")
        row0 = s * ROWS_PER_TILE
        cbase = jnp.where(c == 0, s * NCT0, N_SUBCORES * NCT0 + s * NCT1)
        lbase0 = cbase * CHUNK
        T = jnp.where(c == 0, NCT0 // 6, NCT1 // 6)

        def idx_fetch(kk, islot):
            pltpu.async_copy(idx_h.at[cbase + kk], idxb.at[islot],
                             sem_idx[islot])

        def idx_wait(islot):
            pltpu.make_async_copy(idx_h.at[0], idxb.at[islot],
                                  sem_idx[islot]).wait()

        def gav_start(kk, b, islot):
            lb = lbase0 + kk * CHUNK
            pltpu.async_copy(e_h.at[pl.ds(lb, CHUNK)], re.at[b], sem_e[b])
            pltpu.async_copy(xi_h.at[idxb.at[islot, 1]], ri.at[b], sem_i[b])
            pltpu.async_copy(xj_h.at[idxb.at[islot, 0]], rj.at[b], sem_j[b])

        def gav_wait(b, islot):
            pltpu.make_async_copy(e_h.at[pl.ds(0, CHUNK)], re.at[b],
                                  sem_e[b]).wait()
            pltpu.make_async_copy(xi_h.at[idxb.at[islot, 1]], ri.at[b],
                                  sem_i[b]).wait()
            pltpu.make_async_copy(xj_h.at[idxb.at[islot, 0]], rj.at[b],
                                  sem_j[b]).wait()

        def scat_start(b, islot):
            pltpu.async_copy(re.at[b], agg_sh.at[idxb.at[islot, 2]],
                             sem_sc[b], add=True)

        def scat_wait(b):
            pltpu.make_async_copy(re.at[b], agg_sh.at[pl.ds(0, CHUNK)],
                                  sem_sc[b]).wait()

        def compute(b):
            def row(i, _):
                for j in range(DP // 16):
                    sl = pl.ds(j * 16, 16)
                    v = re[b, i, sl] + ri[b, i, sl] + rj[b, i, sl]
                    re[b, i, sl] = jnp.maximum(v, 0.0)
                return 0
            jax.lax.fori_loop(0, CHUNK, row, 0)

        idx_fetch(0, 0)
        idx_fetch(1, 1)

        def zrow(i, _):
            for j in range(DP // 16):
                re[0, i, pl.ds(j * 16, 16)] = jnp.zeros((16,), _f32)
            return 0
        jax.lax.fori_loop(0, CHUNK, zrow, 0)
        for off, sz in _COPY_CHUNKS:
            pltpu.sync_copy(re.at[0, pl.ds(0, sz)],
                            agg_sh.at[pl.ds(row0 + off, sz)])
        plsc.subcore_barrier()

        idx_wait(0)
        gav_start(0, 0, 0)

        def body6(t, _):
            k0 = t * 6
            for u in range(6):
                kk = k0 + u
                b, o = u % 2, 1 - u % 2
                icur, inxt, ipre = u % 3, (u + 1) % 3, (u + 2) % 3
                gav_wait(b, icur)

                def launch():
                    idx_wait(inxt)
                    if u == 0:
                        @pl.when(t > 0)
                        def _():
                            scat_wait(o)
                    else:
                        scat_wait(o)
                    gav_start(kk + 1, o, inxt)
                if u < 5:
                    launch()
                else:
                    @pl.when(t < T - 1)
                    def _():
                        launch()

                if u < 4:
                    idx_fetch(kk + 2, ipre)
                else:
                    @pl.when(t < T - 1)
                    def _():
                        idx_fetch(kk + 2, ipre)

                compute(b)
                scat_start(b, icur)
            return 0

        jax.lax.fori_loop(0, T, body6, 0)
        scat_wait(0)
        scat_wait(1)
        plsc.subcore_barrier()

        obase = c * NODE_PAD + row0
        for off, sz in _COPY_CHUNKS:
            pltpu.sync_copy(agg_sh.at[pl.ds(row0 + off, sz)],
                            re.at[0, pl.ds(0, sz)])
            pltpu.sync_copy(re.at[0, pl.ds(0, sz)],
                            out_h.at[pl.ds(obase + off, sz)])

    return k(xi_g, xj_g, e_g, idx_il)


def _padw(w, r, c):
    return jnp.zeros((r, c), _f32).at[:w.shape[0], :w.shape[1]].set(w)


def _padb(b):
    return jnp.zeros((8, DP), _f32).at[0, :b.shape[0]].set(b)


def _padidx(a, fill):
    return jnp.full((E_PAD,), fill, jnp.int32).at[:N_EDGES].set(a)


def kernel(node_features_0, node_features_1, edge_features_0, edge_features_1,
           lin1_W, lin1_b, mlp_W1, mlp_b1, mlp_W2, mlp_b2,
           lin2_W, lin2_b, mlp2_W1, mlp2_b1, mlp2_W2, mlp2_b2,
           ntn_W, ntn_V, ntn_b, rule_table, attn_W, gate_W, gate_b,
           fc1_W, fc1_b, fc2_W, fc2_b, fc3_W, fc3_b,
           edge_indices_0, edge_indices_1, rules, ori_lengths):
    x_g0 = jnp.zeros((NODE_PAD, DP), _f32).at[:N_NODES, :D].set(
        node_features_0)
    x_g1 = jnp.zeros((NODE_PAD, DP), _f32).at[:N_NODES, :D].set(
        node_features_1)
    ea_bf0 = jnp.pad(edge_features_0, ((0, 0), (0, DP - D))).astype(_bf16)
    ea_bf1 = jnp.pad(edge_features_1, ((0, 0), (0, DP - D))).astype(_bf16)

    wi1_t = _padw(lin1_W[:, 0:D].T, DP, DP).astype(_bf16)
    we1_t = _padw(lin1_W[:, D:2 * D].T, DP, DP).astype(_bf16)
    wj1_t = _padw(lin1_W[:, 2 * D:3 * D].T, DP, DP).astype(_bf16)
    b1 = _padb(lin1_b)
    wi2_t = _padw(lin2_W[:, 0:D].T, DP, DP).astype(_bf16)
    we2_t = _padw(lin2_W[:, D:2 * D].T, DP, DP).astype(_bf16)
    wj2_t = _padw(lin2_W[:, 2 * D:3 * D].T, DP, DP).astype(_bf16)
    b2 = _padb(lin2_b)
    mw1_t = _padw(mlp_W1.T, DP, DP).astype(_bf16)
    mb1 = _padb(mlp_b1)
    mw2_t = _padw(mlp_W2.T, DP, DP).astype(_bf16)
    mb2 = _padb(mlp_b2)
    m2w1_t = _padw(mlp2_W1.T, DP, DP).astype(_bf16)
    m2b1 = _padb(mlp2_b1)
    m2w2_t = _padw(mlp2_W2.T, DP, DP).astype(_bf16)
    m2b2 = _padb(mlp2_b2)

    def mk_idx(ei):
        src, dst = ei[0], ei[1]
        shp = (N_TILES, CHUNKS_PER_TILE, CHUNK)
        return jnp.stack(
            [_padidx(src, DUMMY).reshape(shp),
             _padidx(dst, DUMMY).reshape(shp),
             _padidx(dst, SCAT_DUMMY).reshape(shp)],
            axis=2).reshape(N_TILES * CHUNKS_PER_TILE, 3, CHUNK)

    idx_g0 = mk_idx(edge_indices_0)
    idx_g1 = mk_idx(edge_indices_1)

    xi1_g0, xj1_g0 = _proj(x_g0.astype(_bf16), wi1_t, wj1_t, b1)
    e1_0 = _edge1(ea_bf0, we1_t)
    agg1_g0 = _sc_conv(xi1_g0, xj1_g0, e1_0, idx_g0)

    xi1_g1, xj1_g1 = _proj(x_g1.astype(_bf16), wi1_t, wj1_t, b1)
    e1_1 = _edge1(ea_bf1, we1_t)
    e2_0 = _edge1(ea_bf0, we2_t)
    agg1_g1 = _sc_conv(xi1_g1, xj1_g1, e1_1, idx_g1)

    x1_g0, xi2_g0, xj2_g0 = _post1(
        agg1_g0, x_g0, mw1_t, mb1, mw2_t, mb2, wi2_t, wj2_t, b2)
    agg2_g0 = _sc_conv(xi2_g0, xj2_g0, e2_0, idx_g0)

    e2_1 = _edge1(ea_bf1, we2_t)
    x1_g1, xi2_g1, xj2_g1 = _post1(
        agg1_g1, x_g1, mw1_t, mb1, mw2_t, mb2, wi2_t, wj2_t, b2)
    agg2_g1 = _sc_conv(xi2_g1, xj2_g1, e2_1, idx_g1)

    cs0 = _post2(agg2_g0, x1_g0, m2w1_t, m2b1, m2w2_t, m2b2)
    cs1 = _post2(agg2_g1, x1_g1, m2w1_t, m2b1, m2w2_t, m2b2)
    g1 = cs0.sum(axis=0)[:DIM]
    g2 = cs1.sum(axis=0)[:DIM]

    bil = jnp.einsum('i,kij,j->k', g1, ntn_W, g2)
    graph_vector = jnp.tanh(bil + ntn_V @ jnp.concatenate([g1, g2]) + ntn_b)
    rule_len = rules.shape[1]
    emb = rule_table[rules]
    mask = (jnp.arange(rule_len)[None, :] < ori_lengths[:, None]).astype(_f32)
    denom = jnp.maximum(ori_lengths, 1).astype(_f32)[:, None]
    rules_embedding = (emb * mask[..., None]).sum(axis=1) / denom
    scores = rules_embedding @ (attn_W @ graph_vector)
    attention_weight = jax.nn.softmax(scores)
    rules_fusion = attention_weight @ rules_embedding
    gate = jax.nn.sigmoid(
        gate_W @ jnp.concatenate([graph_vector, rules_fusion]) + gate_b)
    final_vector = gate * graph_vector + (1.0 - gate) * rules_fusion
    x = jax.nn.relu(fc1_W @ final_vector + fc1_b)
    x = jax.nn.relu(fc2_W @ x + fc2_b)
    x = fc3_W @ x + fc3_b
    return (jnp.abs(x), attention_weight)


# speedup vs baseline: 1.3108x; 1.0040x over previous
import functools

import jax
import jax.numpy as jnp
from jax.experimental import pallas as pl
from jax.experimental.pallas import tpu as pltpu
from jax.experimental.pallas import tpu_sc as plsc

N_NODES = 10000
N_EDGES = 320000
D = 100
DIM = 64

DP = 128
NODE_PAD = 10240
DUMMY = NODE_PAD - 1
N_SUBCORES = 16
N_TILES = 32
CHUNK = 56
CHUNKS_PER_TILE = 180
NCT0 = 222
NCT1 = 138
E_PAD = N_TILES * CHUNK * CHUNKS_PER_TILE
SPMEM_ROWS = 10112
SCAT_DUMMY = N_NODES
ROWS_PER_TILE = SPMEM_ROWS // N_SUBCORES
_COPY_CHUNKS = [(i * CHUNK, CHUNK) for i in range(11)] + [(616, 16)]

_f32 = jnp.float32
_bf16 = jnp.bfloat16


def _dot(a, b):
    return jax.lax.dot_general(a, b, (((1,), (0,)), ((), ())),
                               preferred_element_type=_f32)


def _proj_body(x_ref, wi_ref, wj_ref, b_ref, xi_ref, xj_ref):
    x = x_ref[...]
    xi_ref[...] = _dot(x, wi_ref[...]) + b_ref[0:1, :]
    xj_ref[...] = _dot(x, wj_ref[...])


def _proj(x_bf, wi_t, wj_t, b_pad):
    n = x_bf.shape[0]
    blk = 256
    return pl.pallas_call(
        _proj_body,
        grid=(n // blk,),
        in_specs=[
            pl.BlockSpec((blk, DP), lambda i: (i, 0)),
            pl.BlockSpec((DP, DP), lambda i: (0, 0)),
            pl.BlockSpec((DP, DP), lambda i: (0, 0)),
            pl.BlockSpec((8, DP), lambda i: (0, 0)),
        ],
        out_specs=[
            pl.BlockSpec((blk, DP), lambda i: (i, 0)),
            pl.BlockSpec((blk, DP), lambda i: (i, 0)),
        ],
        out_shape=[
            jax.ShapeDtypeStruct((n, DP), _f32),
            jax.ShapeDtypeStruct((n, DP), _f32),
        ],
    )(x_bf, wi_t, wj_t, b_pad)


def _edge1_body(ea_ref, w_ref, e_ref):
    e_ref[...] = _dot(ea_ref[...].astype(_bf16), w_ref[...])


def _edge1(ea_bf, w_t_bf):
    blk = 1280
    nblk_in = N_EDGES // blk
    return pl.pallas_call(
        _edge1_body,
        grid=(E_PAD // blk,),
        in_specs=[
            pl.BlockSpec((blk, D), lambda i: (jnp.minimum(i, nblk_in - 1), 0)),
            pl.BlockSpec((D, DP), lambda i: (0, 0)),
        ],
        out_specs=pl.BlockSpec((blk, DP), lambda i: (i, 0)),
        out_shape=jax.ShapeDtypeStruct((E_PAD, DP), _f32),
    )(ea_bf, w_t_bf)


def _post1_body(agg_ref, aggb_ref, x_ref, mw1_ref, mb1_ref, mw2_ref, mb2_ref,
                wi2_ref, wj2_ref, b2_ref, x1_ref, xi2_ref, xj2_ref):
    h = (agg_ref[...] + aggb_ref[...] + x_ref[...]).astype(_bf16)
    t = jax.nn.relu(_dot(h, mw1_ref[...]) + mb1_ref[0:1, :])
    x1 = jax.nn.relu(_dot(t.astype(_bf16), mw2_ref[...]) + mb2_ref[0:1, :])
    x1_ref[...] = x1
    x1b = x1.astype(_bf16)
    xi2_ref[...] = _dot(x1b, wi2_ref[...]) + b2_ref[0:1, :]
    xj2_ref[...] = _dot(x1b, wj2_ref[...])


def _post1(agg_pair, x_g, mw1_t, mb1, mw2_t, mb2, wi2_t, wj2_t, b2):
    n = x_g.shape[0]
    blk = 256
    nb = NODE_PAD // blk
    wspec = pl.BlockSpec((DP, DP), lambda i: (0, 0))
    bspec = pl.BlockSpec((8, DP), lambda i: (0, 0))
    rspec = pl.BlockSpec((blk, DP), lambda i: (i, 0))
    pspec = pl.BlockSpec((blk, DP), lambda i: (nb + i, 0))
    return pl.pallas_call(
        _post1_body,
        grid=(n // blk,),
        in_specs=[rspec, pspec, rspec, wspec, bspec, wspec, bspec, wspec,
                  wspec, bspec],
        out_specs=[rspec, rspec, rspec],
        out_shape=[
            jax.ShapeDtypeStruct((n, DP), _f32),
            jax.ShapeDtypeStruct((n, DP), _f32),
            jax.ShapeDtypeStruct((n, DP), _f32),
        ],
    )(agg_pair, agg_pair, x_g, mw1_t, mb1, mw2_t, mb2, wi2_t, wj2_t, b2)


def _post2_body(agg_ref, aggb_ref, x1_ref, mw1_ref, mb1_ref, mw2_ref,
                mb2_ref, out_ref):
    i = pl.program_id(0)
    blk = agg_ref.shape[0]
    h = (agg_ref[...] + aggb_ref[...] + x1_ref[...]).astype(_bf16)
    t = jax.nn.relu(_dot(h, mw1_ref[...]) + mb1_ref[0:1, :])
    x2 = jax.nn.relu(_dot(t.astype(_bf16), mw2_ref[...]) + mb2_ref[0:1, :])
    local_row = i * blk + jax.lax.broadcasted_iota(jnp.int32, (blk, 1), 0)
    x2 = jnp.where(local_row < N_NODES, x2, 0.0)
    part = x2.reshape(blk // 8, 8, DP).sum(axis=0)

    @pl.when(i == 0)
    def _():
        out_ref[...] = jnp.zeros_like(out_ref)

    out_ref[...] += part


def _post2(agg_pair, x1_g, mw1_t, mb1, mw2_t, mb2):
    n = x1_g.shape[0]
    blk = 256
    nb = NODE_PAD // blk
    wspec = pl.BlockSpec((DP, DP), lambda i: (0, 0))
    bspec = pl.BlockSpec((8, DP), lambda i: (0, 0))
    rspec = pl.BlockSpec((blk, DP), lambda i: (i, 0))
    pspec = pl.BlockSpec((blk, DP), lambda i: (nb + i, 0))
    return pl.pallas_call(
        _post2_body,
        grid=(n // blk,),
        in_specs=[rspec, pspec, rspec, wspec, bspec, wspec, bspec],
        out_specs=pl.BlockSpec((8, DP), lambda i: (0, 0)),
        out_shape=jax.ShapeDtypeStruct((8, DP), _f32),
    )(agg_pair, agg_pair, x1_g, mw1_t, mb1, mw2_t, mb2)


def _sc_conv(xi_g, xj_g, e_g, idx_il):
    mesh = plsc.VectorSubcoreMesh(core_axis_name="c", subcore_axis_name="s")
    nct = CHUNKS_PER_TILE

    @functools.partial(
        pl.kernel, mesh=mesh,
        out_type=jax.ShapeDtypeStruct((2 * NODE_PAD, DP), _f32),
        scratch_types=[
            pltpu.VMEM_SHARED((SPMEM_ROWS, DP), _f32),
            pltpu.VMEM((3, 3, CHUNK), jnp.int32),
            pltpu.VMEM((2, CHUNK, DP), _f32),
            pltpu.VMEM((2, CHUNK, DP), _f32),
            pltpu.VMEM((2, CHUNK, DP), _f32),
        ] + [pltpu.SemaphoreType.DMA] * 11,
    )
    def k(xi_h, xj_h, e_h, idx_h, out_h,
          agg_sh, idxb, ri, rj, re,
          sx0, sx1, sx2, si0, si1, sj0, sj1, se0, se1, ss0, ss1):
        sem_idx = [sx0, sx1, sx2]
        sem_i = [si0, si1]
        sem_j = [sj0, sj1]
        sem_e = [se0, se1]
        sem_sc = [ss0, ss1]
        c = jax.lax.axis_index("c")
        s = jax.lax.axis_index("s")
        row0 = s * ROWS_PER_TILE
        cbase = jnp.where(c == 0, s * NCT0, N_SUBCORES * NCT0 + s * NCT1)
        lbase0 = cbase * CHUNK
        T = jnp.where(c == 0, NCT0 // 6, NCT1 // 6)

        def idx_fetch(kk, islot):
            pltpu.async_copy(idx_h.at[cbase + kk], idxb.at[islot],
                             sem_idx[islot])

        def idx_wait(islot):
            pltpu.make_async_copy(idx_h.at[0], idxb.at[islot],
                                  sem_idx[islot]).wait()

        def gav_start(kk, b, islot):
            lb = lbase0 + kk * CHUNK
            pltpu.async_copy(e_h.at[pl.ds(lb, CHUNK)], re.at[b], sem_e[b])
            pltpu.async_copy(xi_h.at[idxb.at[islot, 1]], ri.at[b], sem_i[b])
            pltpu.async_copy(xj_h.at[idxb.at[islot, 0]], rj.at[b], sem_j[b])

        def gav_wait(b, islot):
            pltpu.make_async_copy(e_h.at[pl.ds(0, CHUNK)], re.at[b],
                                  sem_e[b]).wait()
            pltpu.make_async_copy(xi_h.at[idxb.at[islot, 1]], ri.at[b],
                                  sem_i[b]).wait()
            pltpu.make_async_copy(xj_h.at[idxb.at[islot, 0]], rj.at[b],
                                  sem_j[b]).wait()

        def scat_start(b, islot):
            pltpu.async_copy(re.at[b], agg_sh.at[idxb.at[islot, 2]],
                             sem_sc[b], add=True)

        def scat_wait(b):
            pltpu.make_async_copy(re.at[b], agg_sh.at[pl.ds(0, CHUNK)],
                                  sem_sc[b]).wait()

        def compute(b):
            def row(i, _):
                for j in range(DP // 16):
                    sl = pl.ds(j * 16, 16)
                    v = re[b, i, sl] + ri[b, i, sl] + rj[b, i, sl]
                    re[b, i, sl] = jnp.maximum(v, 0.0)
                return 0
            jax.lax.fori_loop(0, CHUNK, row, 0)

        idx_fetch(0, 0)
        idx_fetch(1, 1)

        def zrow(i, _):
            for j in range(DP // 16):
                re[0, i, pl.ds(j * 16, 16)] = jnp.zeros((16,), _f32)
            return 0
        jax.lax.fori_loop(0, CHUNK, zrow, 0)
        for off, sz in _COPY_CHUNKS:
            pltpu.sync_copy(re.at[0, pl.ds(0, sz)],
                            agg_sh.at[pl.ds(row0 + off, sz)])
        plsc.subcore_barrier()

        idx_wait(0)
        gav_start(0, 0, 0)

        def body6(t, _):
            k0 = t * 6
            for u in range(6):
                kk = k0 + u
                b, o = u % 2, 1 - u % 2
                icur, inxt, ipre = u % 3, (u + 1) % 3, (u + 2) % 3
                gav_wait(b, icur)

                def launch():
                    idx_wait(inxt)
                    if u == 0:
                        @pl.when(t > 0)
                        def _():
                            scat_wait(o)
                    else:
                        scat_wait(o)
                    gav_start(kk + 1, o, inxt)
                if u < 5:
                    launch()
                else:
                    @pl.when(t < T - 1)
                    def _():
                        launch()

                if u < 4:
                    idx_fetch(kk + 2, ipre)
                else:
                    @pl.when(t < T - 1)
                    def _():
                        idx_fetch(kk + 2, ipre)

                compute(b)
                scat_start(b, icur)
            return 0

        jax.lax.fori_loop(0, T, body6, 0)
        scat_wait(0)
        scat_wait(1)
        plsc.subcore_barrier()

        obase = c * NODE_PAD + row0
        for off, sz in _COPY_CHUNKS:
            pltpu.sync_copy(agg_sh.at[pl.ds(row0 + off, sz)],
                            re.at[0, pl.ds(0, sz)])
            pltpu.sync_copy(re.at[0, pl.ds(0, sz)],
                            out_h.at[pl.ds(obase + off, sz)])

    return k(xi_g, xj_g, e_g, idx_il)


def _padw(w, r, c):
    return jnp.zeros((r, c), _f32).at[:w.shape[0], :w.shape[1]].set(w)


def _padb(b):
    return jnp.zeros((8, DP), _f32).at[0, :b.shape[0]].set(b)


def _padidx(a, fill):
    return jnp.full((E_PAD,), fill, jnp.int32).at[:N_EDGES].set(a)


def kernel(node_features_0, node_features_1, edge_features_0, edge_features_1,
           lin1_W, lin1_b, mlp_W1, mlp_b1, mlp_W2, mlp_b2,
           lin2_W, lin2_b, mlp2_W1, mlp2_b1, mlp2_W2, mlp2_b2,
           ntn_W, ntn_V, ntn_b, rule_table, attn_W, gate_W, gate_b,
           fc1_W, fc1_b, fc2_W, fc2_b, fc3_W, fc3_b,
           edge_indices_0, edge_indices_1, rules, ori_lengths):
    x_g0 = jnp.zeros((NODE_PAD, DP), _f32).at[:N_NODES, :D].set(
        node_features_0)
    x_g1 = jnp.zeros((NODE_PAD, DP), _f32).at[:N_NODES, :D].set(
        node_features_1)
    ea_bf0 = edge_features_0
    ea_bf1 = edge_features_1

    wi1_t = _padw(lin1_W[:, 0:D].T, DP, DP).astype(_bf16)
    we1_t = _padw(lin1_W[:, D:2 * D].T, D, DP).astype(_bf16)
    wj1_t = _padw(lin1_W[:, 2 * D:3 * D].T, DP, DP).astype(_bf16)
    b1 = _padb(lin1_b)
    wi2_t = _padw(lin2_W[:, 0:D].T, DP, DP).astype(_bf16)
    we2_t = _padw(lin2_W[:, D:2 * D].T, D, DP).astype(_bf16)
    wj2_t = _padw(lin2_W[:, 2 * D:3 * D].T, DP, DP).astype(_bf16)
    b2 = _padb(lin2_b)
    mw1_t = _padw(mlp_W1.T, DP, DP).astype(_bf16)
    mb1 = _padb(mlp_b1)
    mw2_t = _padw(mlp_W2.T, DP, DP).astype(_bf16)
    mb2 = _padb(mlp_b2)
    m2w1_t = _padw(mlp2_W1.T, DP, DP).astype(_bf16)
    m2b1 = _padb(mlp2_b1)
    m2w2_t = _padw(mlp2_W2.T, DP, DP).astype(_bf16)
    m2b2 = _padb(mlp2_b2)

    def mk_idx(ei):
        src, dst = ei[0], ei[1]
        shp = (N_TILES, CHUNKS_PER_TILE, CHUNK)
        return jnp.stack(
            [_padidx(src, DUMMY).reshape(shp),
             _padidx(dst, DUMMY).reshape(shp),
             _padidx(dst, SCAT_DUMMY).reshape(shp)],
            axis=2).reshape(N_TILES * CHUNKS_PER_TILE, 3, CHUNK)

    idx_g0 = mk_idx(edge_indices_0)
    idx_g1 = mk_idx(edge_indices_1)

    xi1_g0, xj1_g0 = _proj(x_g0.astype(_bf16), wi1_t, wj1_t, b1)
    e1_0 = _edge1(ea_bf0, we1_t)
    agg1_g0 = _sc_conv(xi1_g0, xj1_g0, e1_0, idx_g0)

    xi1_g1, xj1_g1 = _proj(x_g1.astype(_bf16), wi1_t, wj1_t, b1)
    e1_1 = _edge1(ea_bf1, we1_t)
    e2_0 = _edge1(ea_bf0, we2_t)
    agg1_g1 = _sc_conv(xi1_g1, xj1_g1, e1_1, idx_g1)

    x1_g0, xi2_g0, xj2_g0 = _post1(
        agg1_g0, x_g0, mw1_t, mb1, mw2_t, mb2, wi2_t, wj2_t, b2)
    agg2_g0 = _sc_conv(xi2_g0, xj2_g0, e2_0, idx_g0)

    e2_1 = _edge1(ea_bf1, we2_t)
    x1_g1, xi2_g1, xj2_g1 = _post1(
        agg1_g1, x_g1, mw1_t, mb1, mw2_t, mb2, wi2_t, wj2_t, b2)
    agg2_g1 = _sc_conv(xi2_g1, xj2_g1, e2_1, idx_g1)

    cs0 = _post2(agg2_g0, x1_g0, m2w1_t, m2b1, m2w2_t, m2b2)
    cs1 = _post2(agg2_g1, x1_g1, m2w1_t, m2b1, m2w2_t, m2b2)
    g1 = cs0.sum(axis=0)[:DIM]
    g2 = cs1.sum(axis=0)[:DIM]

    bil = jnp.einsum('i,kij,j->k', g1, ntn_W, g2)
    graph_vector = jnp.tanh(bil + ntn_V @ jnp.concatenate([g1, g2]) + ntn_b)
    rule_len = rules.shape[1]
    emb = rule_table[rules]
    mask = (jnp.arange(rule_len)[None, :] < ori_lengths[:, None]).astype(_f32)
    denom = jnp.maximum(ori_lengths, 1).astype(_f32)[:, None]
    rules_embedding = (emb * mask[..., None]).sum(axis=1) / denom
    scores = rules_embedding @ (attn_W @ graph_vector)
    attention_weight = jax.nn.softmax(scores)
    rules_fusion = attention_weight @ rules_embedding
    gate = jax.nn.sigmoid(
        gate_W @ jnp.concatenate([graph_vector, rules_fusion]) + gate_b)
    final_vector = gate * graph_vector + (1.0 - gate) * rules_fusion
    x = jax.nn.relu(fc1_W @ final_vector + fc1_b)
    x = jax.nn.relu(fc2_W @ x + fc2_b)
    x = fc3_W @ x + fc3_b
    return (jnp.abs(x), attention_weight)
